# baseline (speedup 1.0000x reference)
import jax
import jax.numpy as jnp
from jax import lax
from jax.experimental import pallas as pl
from jax.experimental.pallas import tpu as pltpu
from jax.experimental.pallas import tpu_sc as plsc

E = 64
CAP = 128
H = 1024
F = 512
T = 2048
NW = 32
SLOTS = E * CAP
EH = E // 2
HSLOTS = EH * CAP
SPW = HSLOTS // NW
TOKS_PER_W = T // NW


def _router_body(x_ref, gw_ref, gb_ref, slot_ref, coef_ref):
    x = x_ref[...]
    logits = jnp.dot(x, gw_ref[...],
                     preferred_element_type=jnp.float32) + gb_ref[...]
    m = jnp.max(logits, axis=1, keepdims=True)
    wgt = 1.0 / jnp.sum(jnp.exp(logits - m), axis=1, keepdims=True)
    eids = lax.broadcasted_iota(jnp.int32, (T, E), 1)
    sel = jnp.min(jnp.where(logits == m, eids, E), axis=1, keepdims=True)
    onehot = (eids == sel).astype(jnp.float32)
    rr = lax.broadcasted_iota(jnp.int32, (T, T), 0)
    cc = lax.broadcasted_iota(jnp.int32, (T, T), 1)
    tril = (rr >= cc).astype(jnp.float32)
    cums = jnp.dot(tril, onehot, preferred_element_type=jnp.float32)
    pos = (jnp.sum(cums * onehot, axis=1, keepdims=True)
           - 1.0).astype(jnp.int32)
    keep = pos < CAP
    slot_ref[...] = jnp.where(keep, sel * CAP + pos, SLOTS)
    coef_ref[...] = jnp.where(keep, wgt, 0.0)


def _router(x, gate_w, gate_b):
    return pl.pallas_call(
        _router_body,
        out_shape=[jax.ShapeDtypeStruct((T, 1), jnp.int32),
                   jax.ShapeDtypeStruct((T, 1), jnp.float32)],
    )(x, gate_w, gate_b)


def _make_dispatch_body(half):
    hbase = half * HSLOTS

    def body(slot_hbm, coef_hbm, x_hbm, eout_hbm, cslot_hbm,
             slots_v, coefs_v, stt_v, cst_v, rows_v, sem):
        cid = lax.axis_index("c")
        sid = lax.axis_index("s")
        wid = sid * 2 + cid
        base = wid * SPW
        pltpu.sync_copy(slot_hbm, slots_v)
        pltpu.sync_copy(coef_hbm, coefs_v)
        lane = lax.broadcasted_iota(jnp.int32, (16,), 0)
        for k in range(SPW // 64):
            for j in range(4):
                stt_v[k, pl.ds(j * 16, 16)] = (
                    (lane + (base + k * 64 + j * 16)) & (T - 1))
        nfill = [jnp.zeros((16,), jnp.int32) for _ in range(SPW // 64)]
        one16 = jnp.ones((16,), jnp.int32)
        zero16 = jnp.zeros((16,), jnp.int32)
        for i in range(T // 16):
            sl = slots_v[pl.ds(i * 16, 16)]
            idx = sl - (hbase + base)
            msk = (idx >= 0) & (idx < SPW)
            chunk = idx >> 6
            plsc.store_scatter(stt_v, [chunk, idx & 63],
                               lane + (i * 16), mask=msk)
            for k in range(SPW // 64):
                nfill[k] = nfill[k] + jnp.where(
                    msk & (chunk == k), one16, zero16)
        for k in range(SPW // 64):
            for j in range(4):
                tix = stt_v[k, pl.ds(j * 16, 16)]
                cst_v[pl.ds(k * 64 + j * 16, 16)] = \
                    plsc.load_gather(coefs_v, [tix])
        pltpu.sync_copy(cst_v, cslot_hbm.at[pl.ds(base, SPW)])
        for k in range(SPW // 64):
            @pl.when(jnp.sum(nfill[k]) > 0)
            def _copy_chunk(k=k):
                pltpu.async_copy(x_hbm.at[stt_v.at[k]], rows_v, sem).wait()
                pltpu.sync_copy(rows_v,
                                eout_hbm.at[pl.ds(base + k * 64, 64)])

    return body


def _dispatch(slot, coef, x, half):
    mesh = plsc.VectorSubcoreMesh(core_axis_name="c", subcore_axis_name="s")
    return pl.kernel(
        _make_dispatch_body(half),
        out_type=[jax.ShapeDtypeStruct((HSLOTS, H), jnp.float32),
                  jax.ShapeDtypeStruct((HSLOTS,), jnp.float32)],
        mesh=mesh,
        scratch_types=[pltpu.VMEM((T,), jnp.int32),
                       pltpu.VMEM((T,), jnp.float32),
                       pltpu.VMEM((SPW // 64, 64), jnp.int32),
                       pltpu.VMEM((SPW,), jnp.float32),
                       pltpu.VMEM((64, H), jnp.float32),
                       pltpu.SemaphoreType.DMA],
        compiler_params=pltpu.CompilerParams(needs_layout_passes=False),
    )(slot, coef, x)


def _ffn_a_body(ein_ref, gpw_ref, upw_ref, dnw_ref, gpb_ref, upb_ref,
                dnb_ref, cs_ref, out_ref):
    e = pl.program_id(0)

    @pl.when(e < EH)
    def _compute():
        xin = ein_ref[...]
        g = jnp.dot(xin, gpw_ref[0],
                    preferred_element_type=jnp.float32) + gpb_ref[0]
        u = jnp.dot(xin, upw_ref[0],
                    preferred_element_type=jnp.float32) + upb_ref[0]
        inter = g * jax.nn.sigmoid(g) * u
        out = jnp.dot(inter, dnw_ref[0],
                      preferred_element_type=jnp.float32) + dnb_ref[0]
        out_ref[...] = out * cs_ref[...]

    @pl.when(e == EH)
    def _zero_bin():
        out_ref[...] = jnp.zeros((CAP, H), jnp.float32)


def _ffn_b_body(prev_ref, ein_ref, gpw_ref, upw_ref, dnw_ref, gpb_ref,
                upb_ref, dnb_ref, cs_ref, out_ref):
    xin = ein_ref[...]
    g = jnp.dot(xin, gpw_ref[0],
                preferred_element_type=jnp.float32) + gpb_ref[0]
    u = jnp.dot(xin, upw_ref[0],
                preferred_element_type=jnp.float32) + upb_ref[0]
    inter = g * jax.nn.sigmoid(g) * u
    out = jnp.dot(inter, dnw_ref[0],
                  preferred_element_type=jnp.float32) + dnb_ref[0]
    out_ref[...] = out * cs_ref[...]


def _ffn_a(ein, gp_w, up_w, down_w, gp_b, up_b, down_b, cslot):
    clamp = lambda e: (jnp.minimum(e, EH - 1), 0)
    clamp3 = lambda e: (jnp.minimum(e, EH - 1), 0, 0)
    return pl.pallas_call(
        _ffn_a_body,
        grid=(EH + 1,),
        in_specs=[
            pl.BlockSpec((CAP, H), clamp),
            pl.BlockSpec((1, H, F), clamp3),
            pl.BlockSpec((1, H, F), clamp3),
            pl.BlockSpec((1, F, H), clamp3),
            pl.BlockSpec((1, 1, F), clamp3),
            pl.BlockSpec((1, 1, F), clamp3),
            pl.BlockSpec((1, 1, H), clamp3),
            pl.BlockSpec((CAP, 1), clamp),
        ],
        out_shape=jax.ShapeDtypeStruct(((E + 1) * CAP, H), jnp.float32),
        out_specs=pl.BlockSpec((CAP, H),
                               lambda e: (jnp.where(e < EH, e, E), 0)),
        compiler_params=pltpu.CompilerParams(
            dimension_semantics=("arbitrary",)),
    )(ein, gp_w, up_w, down_w, gp_b, up_b, down_b, cslot)


def _ffn_b(prev, ein, gp_w, up_w, down_w, gp_b, up_b, down_b, cslot):
    em = lambda e: (e, 0)
    em3 = lambda e: (e + EH, 0, 0)
    return pl.pallas_call(
        _ffn_b_body,
        grid=(EH,),
        in_specs=[
            pl.BlockSpec((8, 128), lambda e: (0, 0)),
            pl.BlockSpec((CAP, H), em),
            pl.BlockSpec((1, H, F), em3),
            pl.BlockSpec((1, H, F), em3),
            pl.BlockSpec((1, F, H), em3),
            pl.BlockSpec((1, 1, F), em3),
            pl.BlockSpec((1, 1, F), em3),
            pl.BlockSpec((1, 1, H), em3),
            pl.BlockSpec((CAP, 1), em),
        ],
        out_shape=jax.ShapeDtypeStruct(((E + 1) * CAP, H), jnp.float32),
        out_specs=pl.BlockSpec((CAP, H), lambda e: (e + EH, 0)),
        input_output_aliases={0: 0},
        compiler_params=pltpu.CompilerParams(
            dimension_semantics=("arbitrary",)),
    )(prev, ein, gp_w, up_w, down_w, gp_b, up_b, down_b, cslot)


def _combine_body(outs_hbm, slot_hbm, fin_hbm, idx_v, rows_v, sem):
    cid = lax.axis_index("c")
    sid = lax.axis_index("s")
    wid = sid * 2 + cid
    base = wid * TOKS_PER_W
    pltpu.sync_copy(slot_hbm.at[pl.ds(base, TOKS_PER_W)], idx_v)
    pltpu.async_copy(outs_hbm.at[idx_v], rows_v, sem).wait()
    pltpu.sync_copy(rows_v, fin_hbm.at[pl.ds(base, TOKS_PER_W)])


def _combine(outs, slot):
    mesh = plsc.VectorSubcoreMesh(core_axis_name="c", subcore_axis_name="s")
    return pl.kernel(
        _combine_body,
        out_type=jax.ShapeDtypeStruct((T, H), jnp.float32),
        mesh=mesh,
        scratch_types=[pltpu.VMEM((TOKS_PER_W,), jnp.int32),
                       pltpu.VMEM((TOKS_PER_W, H), jnp.float32),
                       pltpu.SemaphoreType.DMA],
        compiler_params=pltpu.CompilerParams(needs_layout_passes=False),
    )(outs, slot)


def kernel(hidden_states, gate_w, gate_b, up_w, up_b, gp_w, gp_b,
           down_w, down_b):
    b, s, h = hidden_states.shape
    x = hidden_states.reshape(T, H)
    slot2, coef2 = _router(x, gate_w, gate_b.reshape(1, E))
    slot = slot2.reshape(T)
    coef = coef2.reshape(T)
    ein_a, cslot_a = _dispatch(slot, coef, x, 0)
    ein_b, cslot_b = _dispatch(slot, coef, x, 1)
    gpb3 = gp_b.reshape(E, 1, F)
    upb3 = up_b.reshape(E, 1, F)
    dnb3 = down_b.reshape(E, 1, H)
    outs_a = _ffn_a(ein_a, gp_w, up_w, down_w, gpb3, upb3, dnb3,
                    cslot_a.reshape(HSLOTS, 1))
    outs = _ffn_b(outs_a, ein_b, gp_w, up_w, down_w, gpb3, upb3, dnb3,
                  cslot_b.reshape(HSLOTS, 1))
    fin = _combine(outs, slot)
    return fin.reshape(b, s, h)


# speedup vs baseline: 1.0534x; 1.0534x over previous
import jax
import jax.numpy as jnp
from jax import lax
from jax.experimental import pallas as pl
from jax.experimental.pallas import tpu as pltpu
from jax.experimental.pallas import tpu_sc as plsc

E = 64
CAP = 128
H = 1024
F = 512
T = 2048
NW = 32
SLOTS = E * CAP
SLOTS_PER_W = SLOTS // NW
TOKS_PER_W = T // NW


def _router_body(x_ref, gw_ref, gb_ref, slot_ref, coef_ref):
    x = x_ref[...]
    logits = jnp.dot(x, gw_ref[...],
                     preferred_element_type=jnp.float32) + gb_ref[...]
    m = jnp.max(logits, axis=1, keepdims=True)
    wgt = 1.0 / jnp.sum(jnp.exp(logits - m), axis=1, keepdims=True)
    eids = lax.broadcasted_iota(jnp.int32, (T, E), 1)
    sel = jnp.min(jnp.where(logits == m, eids, E), axis=1, keepdims=True)
    onehot = (eids == sel).astype(jnp.float32)
    rr = lax.broadcasted_iota(jnp.int32, (T, T), 0)
    cc = lax.broadcasted_iota(jnp.int32, (T, T), 1)
    tril = (rr >= cc).astype(jnp.float32)
    cums = jnp.dot(tril, onehot, preferred_element_type=jnp.float32)
    pos = (jnp.sum(cums * onehot, axis=1, keepdims=True)
           - 1.0).astype(jnp.int32)
    keep = pos < CAP
    slot_ref[...] = jnp.where(keep, sel * CAP + pos, SLOTS)
    coef_ref[...] = jnp.where(keep, wgt, 0.0)


def _router(x, gate_w, gate_b):
    return pl.pallas_call(
        _router_body,
        out_shape=[jax.ShapeDtypeStruct((T, 1), jnp.int32),
                   jax.ShapeDtypeStruct((T, 1), jnp.float32)],
    )(x, gate_w, gate_b)


def _dispatch_body(slot_hbm, coef_hbm, x_hbm, eout_hbm, cslot_hbm,
                   slots_v, coefs_v, stt_v, cst_v, rows_v, sem):
    cid = lax.axis_index("c")
    sid = lax.axis_index("s")
    wid = sid * 2 + cid
    base = wid * SLOTS_PER_W
    pltpu.sync_copy(slot_hbm, slots_v)
    pltpu.sync_copy(coef_hbm, coefs_v)
    lane = lax.broadcasted_iota(jnp.int32, (16,), 0)
    for k in range(SLOTS_PER_W // 64):
        for j in range(4):
            stt_v[k, pl.ds(j * 16, 16)] = (
                (lane + (base + k * 64 + j * 16)) & (T - 1))
    nfill = [jnp.zeros((16,), jnp.int32) for _ in range(4)]
    one16 = jnp.ones((16,), jnp.int32)
    zero16 = jnp.zeros((16,), jnp.int32)
    for i in range(T // 16):
        sl = slots_v[pl.ds(i * 16, 16)]
        idx = sl - base
        msk = (idx >= 0) & (idx < SLOTS_PER_W)
        chunk = idx >> 6
        plsc.store_scatter(stt_v, [chunk, idx & 63],
                           lane + (i * 16), mask=msk)
        for k in range(4):
            nfill[k] = nfill[k] + jnp.where(
                msk & (chunk == k), one16, zero16)
    for k in range(SLOTS_PER_W // 64):
        for j in range(4):
            tix = stt_v[k, pl.ds(j * 16, 16)]
            cst_v[pl.ds(k * 64 + j * 16, 16)] = \
                plsc.load_gather(coefs_v, [tix])
    pltpu.sync_copy(cst_v, cslot_hbm.at[pl.ds(base, SLOTS_PER_W)])
    for k in range(SLOTS_PER_W // 64):
        @pl.when(jnp.sum(nfill[k]) > 0)
        def _copy_chunk(k=k):
            pltpu.async_copy(x_hbm.at[stt_v.at[k]], rows_v, sem).wait()
            pltpu.sync_copy(rows_v,
                            eout_hbm.at[pl.ds(base + k * 64, 64)])


def _dispatch(slot, coef, x):
    mesh = plsc.VectorSubcoreMesh(core_axis_name="c", subcore_axis_name="s")
    return pl.kernel(
        _dispatch_body,
        out_type=[jax.ShapeDtypeStruct((SLOTS, H), jnp.float32),
                  jax.ShapeDtypeStruct((SLOTS,), jnp.float32)],
        mesh=mesh,
        scratch_types=[pltpu.VMEM((T,), jnp.int32),
                       pltpu.VMEM((T,), jnp.float32),
                       pltpu.VMEM((SLOTS_PER_W // 64, 64), jnp.int32),
                       pltpu.VMEM((SLOTS_PER_W,), jnp.float32),
                       pltpu.VMEM((64, H), jnp.float32),
                       pltpu.SemaphoreType.DMA],
        compiler_params=pltpu.CompilerParams(needs_layout_passes=False),
    )(slot, coef, x)


def _ffn_body(ein_ref, gpw_ref, upw_ref, dnw_ref, gpb_ref, upb_ref, dnb_ref,
              cs_ref, out_ref):
    e = pl.program_id(0)

    @pl.when(e < E)
    def _compute():
        xin = ein_ref[...]
        g = jnp.dot(xin, gpw_ref[0],
                    preferred_element_type=jnp.float32) + gpb_ref[0]
        u = jnp.dot(xin, upw_ref[0],
                    preferred_element_type=jnp.float32) + upb_ref[0]
        inter = g * jax.nn.sigmoid(g) * u
        out = jnp.dot(inter, dnw_ref[0],
                      preferred_element_type=jnp.float32) + dnb_ref[0]
        out_ref[...] = out * cs_ref[...]

    @pl.when(e == E)
    def _zero_bin():
        out_ref[...] = jnp.zeros((CAP, H), jnp.float32)


def _ffn(ein, gp_w, up_w, down_w, gp_b, up_b, down_b, cslot):
    clamp = lambda e: (jnp.minimum(e, E - 1), 0)
    clamp3 = lambda e: (jnp.minimum(e, E - 1), 0, 0)
    return pl.pallas_call(
        _ffn_body,
        grid=(E + 1,),
        in_specs=[
            pl.BlockSpec((CAP, H), clamp),
            pl.BlockSpec((1, H, F), clamp3),
            pl.BlockSpec((1, H, F), clamp3),
            pl.BlockSpec((1, F, H), clamp3),
            pl.BlockSpec((1, 1, F), clamp3),
            pl.BlockSpec((1, 1, F), clamp3),
            pl.BlockSpec((1, 1, H), clamp3),
            pl.BlockSpec((CAP, 1), clamp),
        ],
        out_shape=jax.ShapeDtypeStruct(((E + 1) * CAP, H), jnp.float32),
        out_specs=pl.BlockSpec((CAP, H), lambda e: (e, 0)),
        compiler_params=pltpu.CompilerParams(
            dimension_semantics=("arbitrary",)),
    )(ein, gp_w, up_w, down_w, gp_b, up_b, down_b, cslot)


def _combine_body(outs_hbm, slot_hbm, fin_hbm, idx_v, rows_v, sem):
    cid = lax.axis_index("c")
    sid = lax.axis_index("s")
    wid = sid * 2 + cid
    base = wid * TOKS_PER_W
    pltpu.sync_copy(slot_hbm.at[pl.ds(base, TOKS_PER_W)], idx_v)
    pltpu.async_copy(outs_hbm.at[idx_v], rows_v, sem).wait()
    pltpu.sync_copy(rows_v, fin_hbm.at[pl.ds(base, TOKS_PER_W)])


def _combine(outs, slot):
    mesh = plsc.VectorSubcoreMesh(core_axis_name="c", subcore_axis_name="s")
    return pl.kernel(
        _combine_body,
        out_type=jax.ShapeDtypeStruct((T, H), jnp.float32),
        mesh=mesh,
        scratch_types=[pltpu.VMEM((TOKS_PER_W,), jnp.int32),
                       pltpu.VMEM((TOKS_PER_W, H), jnp.float32),
                       pltpu.SemaphoreType.DMA],
        compiler_params=pltpu.CompilerParams(needs_layout_passes=False),
    )(outs, slot)


def kernel(hidden_states, gate_w, gate_b, up_w, up_b, gp_w, gp_b,
           down_w, down_b):
    b, s, h = hidden_states.shape
    x = hidden_states.reshape(T, H)
    slot2, coef2 = _router(x, gate_w, gate_b.reshape(1, E))
    slot = slot2.reshape(T)
    coef = coef2.reshape(T)
    ein, cslot = _dispatch(slot, coef, x)
    outs = _ffn(ein, gp_w, up_w, down_w, gp_b.reshape(E, 1, F),
                up_b.reshape(E, 1, F), down_b.reshape(E, 1, H),
                cslot.reshape(SLOTS, 1))
    fin = _combine(outs, slot)
    return fin.reshape(b, s, h)
